# baseline (speedup 1.0000x reference)
import functools

import jax
import jax.numpy as jnp
from jax import lax
from jax.experimental import pallas as pl
from jax.experimental.pallas import tpu as pltpu
from jax.experimental.pallas import tpu_sc as plsc

_SHAPE = (64, 32768)
_N = _SHAPE[0] * _SHAPE[1]
_K = int(0.1 * _N)
_NBINS = 2048
_NW = 32
_CHUNK = _N // _NW
_ITERS = _CHUNK // 16

@functools.cache
def _make_hist_kernel():
    mesh = plsc.VectorSubcoreMesh(core_axis_name="c", subcore_axis_name="s")

    @functools.partial(
        pl.kernel,
        mesh=mesh,
        compiler_params=pltpu.CompilerParams(
            needs_layout_passes=False, use_tc_tiling_on_sc=False),
        out_type=jax.ShapeDtypeStruct((_NW, _NBINS), jnp.int32),
        scratch_types=[
            pltpu.VMEM((_CHUNK,), jnp.int32),
            pltpu.VMEM((_NBINS, 16), jnp.int32),
            pltpu.VMEM((_NBINS,), jnp.int32),
            pltpu.VMEM((4, 16), jnp.int32),
        ],
    )
    def _hist_kernel(x_hbm, par_hbm, out_hbm, data_v, hist_v, red_v, par_v):
        cid = lax.axis_index("c")
        sid = lax.axis_index("s")
        wid = sid * 2 + cid
        base = wid * _CHUNK
        pltpu.sync_copy(x_hbm.at[pl.ds(base, _CHUNK)], data_v)
        pltpu.sync_copy(par_hbm, par_v)
        ps = par_v[0, :]
        pv = par_v[1, :]
        bs = par_v[2, :]
        bm = par_v[3, :]
        lane = lax.iota(jnp.int32, 16)
        zeros = jnp.zeros((16,), jnp.int32)
        ones = jnp.full((16,), 1, jnp.int32)

        def zero_body(i, carry):
            base_i = i * 8
            for j in range(8):
                hist_v[base_i + j, :] = zeros
            return carry

        lax.fori_loop(0, _NBINS // 8, zero_body, 0)

        def zero_red(i, carry):
            red_v[pl.ds(i * 16, 16)] = zeros
            return carry

        lax.fori_loop(0, _NBINS // 16, zero_red, 0)

        def body(i, carry):
            base_i = i * 128
            for j in range(8):
                u = data_v[pl.ds(base_i + j * 16, 16)] & jnp.int32(0x7FFFFFFF)
                m = lax.shift_right_logical(u, ps) == pv
                bins = lax.shift_right_logical(u, bs) & bm
                plsc.addupdate_scatter(hist_v, [bins, lane], ones, mask=m)
            return carry

        lax.fori_loop(0, _ITERS // 8, body, 0)

        def red_body(i, carry):
            base_i = i * 8
            for j in range(8):
                b = base_i + j
                bvec = jnp.full((16,), b, jnp.int32)
                plsc.addupdate_scatter(red_v, [bvec], hist_v[b, :])
            return carry

        lax.fori_loop(0, _NBINS // 8, red_body, 0)
        pltpu.sync_copy(red_v, out_hbm.at[wid])

    return _hist_kernel


def _params(ps, pv, bs, bm):
    return jnp.stack([
        jnp.full((16,), ps, jnp.int32),
        jnp.full((16,), pv, jnp.int32),
        jnp.full((16,), bs, jnp.int32),
        jnp.full((16,), bm, jnp.int32),
    ])


def _find_bucket(h, k):
    desc = jnp.cumsum(h[::-1])[::-1]
    idx = jnp.arange(_NBINS, dtype=jnp.int32)
    b = jnp.max(jnp.where(desc >= k, idx, jnp.int32(-1)))
    k_next = k - (desc[b] - h[b])
    return b, k_next


def _mask_body(t_ref, x_ref, o_ref):
    t = t_ref[0]
    xf = x_ref[...]
    u = lax.bitcast_convert_type(xf, jnp.int32) & jnp.int32(0x7FFFFFFF)
    o_ref[...] = jnp.where(u >= t, xf, 0.0)


def kernel(x):
    hist = _make_hist_kernel()
    xu = lax.bitcast_convert_type(x.reshape(-1), jnp.int32)
    h1 = hist(xu, _params(31, 0, 20, 2047)).sum(axis=0)
    b1, k2 = _find_bucket(h1, _K)
    h2 = hist(xu, _params(20, b1, 9, 2047)).sum(axis=0)
    b2, k3 = _find_bucket(h2, k2)
    pre2 = (b1 << 11) | b2
    h3 = hist(xu, _params(9, pre2, 0, 511)).sum(axis=0)
    b3, _ = _find_bucket(h3, k3)
    t = (pre2 << 9) | b3
    t_arr = jnp.reshape(t, (1,)).astype(jnp.int32)
    return pl.pallas_call(
        _mask_body,
        grid=(8,),
        in_specs=[
            pl.BlockSpec(memory_space=pltpu.SMEM),
            pl.BlockSpec((8, 32768), lambda i: (i, 0)),
        ],
        out_specs=pl.BlockSpec((8, 32768), lambda i: (i, 0)),
        out_shape=jax.ShapeDtypeStruct(_SHAPE, jnp.float32),
    )(t_arr, x)


# speedup vs baseline: 1.3147x; 1.3147x over previous
import functools

import jax
import jax.numpy as jnp
from jax import lax
from jax.experimental import pallas as pl
from jax.experimental.pallas import tpu as pltpu
from jax.experimental.pallas import tpu_sc as plsc

_SHAPE = (64, 32768)
_N = _SHAPE[0] * _SHAPE[1]
_K = int(0.1 * _N)
_NBINS = 2048
_NW = 32
_CHUNK = _N // _NW
_ITERS = _CHUNK // 16

@functools.cache
def _make_hist_kernel():
    mesh = plsc.VectorSubcoreMesh(core_axis_name="c", subcore_axis_name="s")

    @functools.partial(
        pl.kernel,
        mesh=mesh,
        compiler_params=pltpu.CompilerParams(
            needs_layout_passes=False, use_tc_tiling_on_sc=False),
        out_type=jax.ShapeDtypeStruct((_NW, _NBINS, 16), jnp.int32),
        scratch_types=[
            pltpu.VMEM((_CHUNK,), jnp.int32),
            pltpu.VMEM((_NBINS, 16), jnp.int32),
            pltpu.VMEM((4, 16), jnp.int32),
        ],
    )
    def _hist_kernel(x_hbm, par_hbm, out_hbm, data_v, hist_v, par_v):
        cid = lax.axis_index("c")
        sid = lax.axis_index("s")
        wid = sid * 2 + cid
        base = wid * _CHUNK
        pltpu.sync_copy(x_hbm.at[pl.ds(base, _CHUNK)], data_v)
        pltpu.sync_copy(par_hbm, par_v)
        ps = par_v[0, :]
        pv = par_v[1, :]
        bs = par_v[2, :]
        bm = par_v[3, :]
        lane = lax.iota(jnp.int32, 16)
        zeros = jnp.zeros((16,), jnp.int32)
        ones = jnp.full((16,), 1, jnp.int32)

        @plsc.parallel_loop(0, _NBINS, step=8, unroll=8)
        def _(i):
            for j in range(8):
                hist_v[i + j, :] = zeros

        @plsc.parallel_loop(0, _ITERS, step=8, unroll=8)
        def _(i):
            base_i = i * 16
            for j in range(8):
                u = data_v[pl.ds(base_i + j * 16, 16)] & jnp.int32(0x7FFFFFFF)
                m = lax.shift_right_logical(u, ps) == pv
                bins = lax.shift_right_logical(u, bs) & bm
                plsc.addupdate_scatter(hist_v, [bins, lane], ones, mask=m)

        pltpu.sync_copy(hist_v, out_hbm.at[wid])

    return _hist_kernel


def _params(ps, pv, bs, bm):
    return jnp.stack([
        jnp.full((16,), ps, jnp.int32),
        jnp.full((16,), pv, jnp.int32),
        jnp.full((16,), bs, jnp.int32),
        jnp.full((16,), bm, jnp.int32),
    ])


def _find_bucket(h, k):
    desc = jnp.cumsum(h[::-1])[::-1]
    idx = jnp.arange(_NBINS, dtype=jnp.int32)
    b = jnp.max(jnp.where(desc >= k, idx, jnp.int32(-1)))
    k_next = k - (desc[b] - h[b])
    return b, k_next


def _mask_body(t_ref, x_ref, o_ref):
    t = t_ref[0]
    xf = x_ref[...]
    u = lax.bitcast_convert_type(xf, jnp.int32) & jnp.int32(0x7FFFFFFF)
    o_ref[...] = jnp.where(u >= t, xf, 0.0)


def kernel(x):
    hist = _make_hist_kernel()
    xu = lax.bitcast_convert_type(x.reshape(-1), jnp.int32)
    h1 = hist(xu, _params(31, 0, 20, 2047)).sum(axis=(0, 2))
    b1, k2 = _find_bucket(h1, _K)
    h2 = hist(xu, _params(20, b1, 9, 2047)).sum(axis=(0, 2))
    b2, k3 = _find_bucket(h2, k2)
    pre2 = (b1 << 11) | b2
    h3 = hist(xu, _params(9, pre2, 0, 511)).sum(axis=(0, 2))
    b3, _ = _find_bucket(h3, k3)
    t = (pre2 << 9) | b3
    t_arr = jnp.reshape(t, (1,)).astype(jnp.int32)
    return pl.pallas_call(
        _mask_body,
        grid=(8,),
        in_specs=[
            pl.BlockSpec(memory_space=pltpu.SMEM),
            pl.BlockSpec((8, 32768), lambda i: (i, 0)),
        ],
        out_specs=pl.BlockSpec((8, 32768), lambda i: (i, 0)),
        out_shape=jax.ShapeDtypeStruct(_SHAPE, jnp.float32),
    )(t_arr, x)


# speedup vs baseline: 2.2120x; 1.6825x over previous
import functools

import jax
import jax.numpy as jnp
from jax import lax
from jax.experimental import pallas as pl
from jax.experimental.pallas import tpu as pltpu
from jax.experimental.pallas import tpu_sc as plsc

_SHAPE = (64, 32768)
_N = _SHAPE[0] * _SHAPE[1]
_K = int(0.1 * _N)
_NBINS = 2048
_HISTW = _NBINS * 16
_NW = 32
_CHUNK = _N // _NW
_ITERS = _CHUNK // 16


@functools.cache
def _make_hist_kernel():
    mesh = plsc.VectorSubcoreMesh(core_axis_name="c", subcore_axis_name="s")

    @functools.partial(
        pl.kernel,
        mesh=mesh,
        compiler_params=pltpu.CompilerParams(
            needs_layout_passes=False, use_tc_tiling_on_sc=False),
        out_type=jax.ShapeDtypeStruct((_NW, _HISTW), jnp.int32),
        scratch_types=[
            pltpu.VMEM((_CHUNK,), jnp.float32),
            pltpu.VMEM((_HISTW,), jnp.int32),
        ],
    )
    def _hist_kernel(x_hbm, out_hbm, data_v, hist_v):
        cid = lax.axis_index("c")
        sid = lax.axis_index("s")
        wid = sid * 2 + cid
        pltpu.sync_copy(x_hbm.at[pl.ds(wid * _CHUNK, _CHUNK)], data_v)
        lane = lax.iota(jnp.int32, 16)
        zeros = jnp.zeros((16,), jnp.int32)
        ones = jnp.full((16,), 1, jnp.int32)

        @plsc.parallel_loop(0, _HISTW // 16, step=8, unroll=8)
        def _(i):
            for j in range(8):
                hist_v[pl.ds((i + j) * 16, 16)] = zeros

        @plsc.parallel_loop(0, _ITERS, step=8, unroll=8)
        def _(i):
            base_i = i * 16
            for j in range(8):
                v = data_v[pl.ds(base_i + j * 16, 16)]
                u = plsc.bitcast(v, jnp.int32) & jnp.int32(0x7FFFFFFF)
                idx = lax.shift_left(lax.shift_right_logical(u, 20), 4) | lane
                plsc.addupdate_scatter(hist_v, [idx], ones)

        pltpu.sync_copy(hist_v, out_hbm.at[wid])

    return _hist_kernel


def _finish_body(p_ref, x_ref, o_ref):
    p = p_ref[...]
    binid = lax.shift_right_logical(
        lax.broadcasted_iota(jnp.int32, (_NW, _HISTW), 1), 4)
    xf = x_ref[...]
    u = lax.bitcast_convert_type(xf, jnp.int32) & jnp.int32(0x7FFFFFFF)

    def hist_step(i, t):
        cand = t | (jnp.int32(1) << (jnp.int32(30) - i))
        cnt = jnp.sum(jnp.where(binid >= lax.shift_right_logical(cand, 20),
                                p, 0))
        return jnp.where(cnt >= _K, cand, t)

    t = lax.fori_loop(0, 11, hist_step, jnp.int32(0))

    def data_step(i, t):
        cand = t | (jnp.int32(1) << (jnp.int32(19) - i))
        cnt = jnp.sum((u >= cand).astype(jnp.int32))
        return jnp.where(cnt >= _K, cand, t)

    t = lax.fori_loop(0, 20, data_step, t)
    o_ref[...] = jnp.where(u >= t, xf, 0.0)


def kernel(x):
    hist = _make_hist_kernel()
    partials = hist(x.reshape(-1))
    return pl.pallas_call(
        _finish_body,
        out_shape=jax.ShapeDtypeStruct(_SHAPE, jnp.float32),
    )(partials, x)


# speedup vs baseline: 2.2717x; 1.0270x over previous
import functools

import jax
import jax.numpy as jnp
from jax import lax
from jax.experimental import pallas as pl
from jax.experimental.pallas import tpu as pltpu
from jax.experimental.pallas import tpu_sc as plsc

_SHAPE = (64, 32768)
_N = _SHAPE[0] * _SHAPE[1]
_K = int(0.1 * _N)
_NBINS = 32768
_NW = 32
_ROWS_PER_W = _SHAPE[0] // _NW
_ROW_ITERS = _SHAPE[1] // 16


@functools.cache
def _make_hist_kernel():
    mesh = plsc.VectorSubcoreMesh(core_axis_name="c", subcore_axis_name="s")

    @functools.partial(
        pl.kernel,
        mesh=mesh,
        compiler_params=pltpu.CompilerParams(
            needs_layout_passes=False, use_tc_tiling_on_sc=False),
        out_type=jax.ShapeDtypeStruct((_NW, _NBINS), jnp.int32),
        scratch_types=[
            pltpu.VMEM((_ROWS_PER_W, _SHAPE[1]), jnp.float32),
            pltpu.VMEM((_NBINS,), jnp.int32),
        ],
    )
    def _hist_kernel(x_hbm, out_hbm, data_v, hist_v):
        cid = lax.axis_index("c")
        sid = lax.axis_index("s")
        wid = sid * 2 + cid
        pltpu.sync_copy(x_hbm.at[pl.ds(wid * _ROWS_PER_W, _ROWS_PER_W)],
                        data_v)
        zeros = jnp.zeros((16,), jnp.int32)
        ones = jnp.full((16,), 1, jnp.int32)

        @plsc.parallel_loop(0, _NBINS // 16, step=8, unroll=8)
        def _(i):
            for j in range(8):
                hist_v[pl.ds((i + j) * 16, 16)] = zeros

        for r in range(_ROWS_PER_W):
            @plsc.parallel_loop(0, _ROW_ITERS, step=8, unroll=8)
            def _(i, r=r):
                base_i = i * 16
                for j in range(8):
                    v = data_v[r, pl.ds(base_i + j * 16, 16)]
                    u = plsc.bitcast(v, jnp.int32) & jnp.int32(0x7FFFFFFF)
                    plsc.addupdate_scatter(
                        hist_v, [lax.shift_right_logical(u, 16)], ones)

        pltpu.sync_copy(hist_v, out_hbm.at[wid])

    return _hist_kernel


def _finish_body(p_ref, x_ref, o_ref):
    p = p_ref[...]
    binid = lax.broadcasted_iota(jnp.int32, (_NW, _NBINS), 1)
    xf = x_ref[...]
    u = lax.bitcast_convert_type(xf, jnp.int32) & jnp.int32(0x7FFFFFFF)

    def hist_step(i, t):
        cand = t | (jnp.int32(1) << (jnp.int32(30) - i))
        cnt = jnp.sum(jnp.where(binid >= lax.shift_right_logical(cand, 16),
                                p, 0))
        return jnp.where(cnt >= _K, cand, t)

    t = lax.fori_loop(0, 15, hist_step, jnp.int32(0))

    def data_step(i, t):
        cand = t | (jnp.int32(1) << (jnp.int32(15) - i))
        cnt = jnp.sum((u >= cand).astype(jnp.int32))
        return jnp.where(cnt >= _K, cand, t)

    t = lax.fori_loop(0, 16, data_step, t)
    o_ref[...] = jnp.where(u >= t, xf, 0.0)


def kernel(x):
    hist = _make_hist_kernel()
    partials = hist(x)
    return pl.pallas_call(
        _finish_body,
        out_shape=jax.ShapeDtypeStruct(_SHAPE, jnp.float32),
    )(partials, x)
